# Initial kernel scaffold; baseline (speedup 1.0000x reference)
import jax
import jax.numpy as jnp
from jax.experimental import pallas as pl


def kernel(x, W_enc, dictionary):
    raise NotImplementedError("write your pallas kernel here")


# speedup vs baseline: 11.3896x; 11.3896x over previous
import functools

import jax
import jax.numpy as jnp
from jax.experimental import pallas as pl
from jax.experimental.pallas import tpu as pltpu

_D_MODEL = 1024
_DICT = 4096
_K = 8
_BLK_M = 128


def _tc_body(x_ref, w_ref, d_ref, off_ref, loss_ref):
    i = pl.program_id(0)
    coeffs = jax.lax.dot_general(
        x_ref[...], w_ref[...],
        dimension_numbers=(((1,), (1,)), ((), ())),
        preferred_element_type=jnp.float32,
    )
    col = jax.lax.broadcasted_iota(jnp.int32, coeffs.shape, 1)
    chosen = jnp.zeros(coeffs.shape, jnp.bool_)
    work = coeffs
    neg = jnp.float32(-jnp.inf)
    for _ in range(_K):
        m = jnp.max(work, axis=1, keepdims=True)
        elig = work == m
        first = jnp.min(jnp.where(elig, col, _DICT), axis=1, keepdims=True)
        pick = col == first
        chosen = jnp.logical_or(chosen, pick)
        work = jnp.where(pick, neg, work)
    sparse = jnp.where(chosen, coeffs, jnp.float32(0.0))
    off_ref[...] = jax.lax.dot_general(
        sparse, d_ref[...],
        dimension_numbers=(((1,), (0,)), ((), ())),
        preferred_element_type=jnp.float32,
    )

    @pl.when(i == 0)
    def _():
        loss_ref[0, 0] = jnp.float32(0.0)

    loss_ref[0, 0] += jnp.sum(jnp.abs(sparse))


@jax.jit
def kernel(x, W_enc, dictionary):
    B, T, D = x.shape
    n_tok = B * T
    xf = x.reshape(n_tok, D)
    grid = n_tok // _BLK_M
    off, loss = pl.pallas_call(
        _tc_body,
        grid=(grid,),
        in_specs=[
            pl.BlockSpec((_BLK_M, _D_MODEL), lambda i: (i, 0)),
            pl.BlockSpec((_DICT, _D_MODEL), lambda i: (0, 0)),
            pl.BlockSpec((_DICT, _D_MODEL), lambda i: (0, 0)),
        ],
        out_specs=[
            pl.BlockSpec((_BLK_M, _D_MODEL), lambda i: (i, 0)),
            pl.BlockSpec(memory_space=pltpu.SMEM),
        ],
        out_shape=[
            jax.ShapeDtypeStruct((n_tok, _D_MODEL), jnp.float32),
            jax.ShapeDtypeStruct((1, 1), jnp.float32),
        ],
        compiler_params=pltpu.CompilerParams(
            dimension_semantics=("arbitrary",),
        ),
    )(xf, W_enc, dictionary)
    offset = off.reshape(B, T, D)
    sparsity_loss = loss[0, 0] / jnp.float32(n_tok * _DICT)
    return (offset, sparsity_loss)


# speedup vs baseline: 18.7110x; 1.6428x over previous
import functools

import jax
import jax.numpy as jnp
from jax.experimental import pallas as pl
from jax.experimental.pallas import tpu as pltpu

_D_MODEL = 1024
_DICT = 4096
_K = 8
_BLK_M = 128


def _tc_body(x_ref, w_ref, d_ref, off_ref, loss_ref):
    i = pl.program_id(0)
    coeffs = jax.lax.dot_general(
        x_ref[...], w_ref[...],
        dimension_numbers=(((1,), (1,)), ((), ())),
        preferred_element_type=jnp.float32,
    )
    work = coeffs
    neg = jnp.float32(-jnp.inf)
    for _ in range(_K):
        m = jnp.max(work, axis=1, keepdims=True)
        work = jnp.where(work == m, neg, work)
    sparse = jnp.where(work == neg, coeffs, jnp.float32(0.0))
    off_ref[...] = jax.lax.dot_general(
        sparse, d_ref[...],
        dimension_numbers=(((1,), (0,)), ((), ())),
        preferred_element_type=jnp.float32,
    )

    @pl.when(i == 0)
    def _():
        loss_ref[0, 0] = jnp.float32(0.0)

    loss_ref[0, 0] += jnp.sum(jnp.abs(sparse))


@jax.jit
def kernel(x, W_enc, dictionary):
    B, T, D = x.shape
    n_tok = B * T
    xf = x.reshape(n_tok, D)
    grid = n_tok // _BLK_M
    off, loss = pl.pallas_call(
        _tc_body,
        grid=(grid,),
        in_specs=[
            pl.BlockSpec((_BLK_M, _D_MODEL), lambda i: (i, 0)),
            pl.BlockSpec((_DICT, _D_MODEL), lambda i: (0, 0)),
            pl.BlockSpec((_DICT, _D_MODEL), lambda i: (0, 0)),
        ],
        out_specs=[
            pl.BlockSpec((_BLK_M, _D_MODEL), lambda i: (i, 0)),
            pl.BlockSpec(memory_space=pltpu.SMEM),
        ],
        out_shape=[
            jax.ShapeDtypeStruct((n_tok, _D_MODEL), jnp.float32),
            jax.ShapeDtypeStruct((1, 1), jnp.float32),
        ],
        compiler_params=pltpu.CompilerParams(
            dimension_semantics=("arbitrary",),
        ),
    )(xf, W_enc, dictionary)
    offset = off.reshape(B, T, D)
    sparsity_loss = loss[0, 0] / jnp.float32(n_tok * _DICT)
    return (offset, sparsity_loss)
